# baseline (speedup 1.0000x reference)
import functools

import jax
import jax.numpy as jnp
from jax import lax
from jax.experimental import pallas as pl
from jax.experimental.pallas import tpu as pltpu
from jax.experimental.pallas import tpu_sc as plsc

D = 64
B = 16384
CHUNK = 128


@functools.cache
def _build():
    info = plsc.get_sparse_core_info()
    nw = info.num_cores * info.num_subcores
    b_per_w = B // nw
    n_chunks = b_per_w // CHUNK
    mesh = plsc.VectorSubcoreMesh(core_axis_name="c", subcore_axis_name="s")

    @functools.partial(
        pl.kernel,
        mesh=mesh,
        out_type=jax.ShapeDtypeStruct((B, D), jnp.float32),
        scratch_types=[
            pltpu.VMEM((b_per_w,), jnp.int32),
            pltpu.VMEM((b_per_w,), jnp.int32),
            pltpu.VMEM((n_chunks, CHUNK, 2 * D), jnp.float32),
            pltpu.VMEM((CHUNK, D), jnp.float32),
            pltpu.SemaphoreType.DMA,
        ],
    )
    def gather_kernel(idx_hbm, pairs_hbm, out_hbm, idx_v, pair_v, bufs,
                      rows_v, sem):
        wid = lax.axis_index("s") * info.num_cores + lax.axis_index("c")
        base = wid * b_per_w
        pltpu.sync_copy(idx_hbm.at[pl.ds(base, b_per_w)], idx_v)

        def split(g, _):
            vec = idx_v[pl.ds(g * 16, 16)]
            pair_v[pl.ds(g * 16, 16)] = lax.shift_right_logical(vec, 1)
            return 0

        lax.fori_loop(0, b_per_w // 16, split, 0)

        copies = [
            pltpu.async_copy(
                pairs_hbm.at[pair_v.at[pl.ds(c * CHUNK, CHUNK)]],
                bufs.at[c],
                sem,
            )
            for c in range(n_chunks)
        ]
        for c, cp in enumerate(copies):
            cp.wait()

            def compact(g, _):
                rem16 = lax.bitwise_and(
                    idx_v[pl.ds(c * CHUNK + g * 16, 16)], 1) * D
                for l in range(16):
                    off = rem16[l]
                    for k in range(D // 16):
                        rows_v[g * 16 + l, pl.ds(k * 16, 16)] = (
                            bufs[c, g * 16 + l, pl.ds(off + k * 16, 16)]
                        )
                return 0

            lax.fori_loop(0, CHUNK // 16, compact, 0)
            pltpu.sync_copy(
                rows_v, out_hbm.at[pl.ds(base + c * CHUNK, CHUNK)]
            )

    return gather_kernel


def kernel(indices, weight):
    pairs = weight.reshape(weight.shape[0] // 2, 2 * D)
    return _build()(indices.astype(jnp.int32), pairs)


# speedup vs baseline: 1.0008x; 1.0008x over previous
import functools

import jax
import jax.numpy as jnp
from jax import lax
from jax.experimental import pallas as pl
from jax.experimental.pallas import tpu as pltpu
from jax.experimental.pallas import tpu_sc as plsc

D = 64
B = 16384
CHUNK = 128


@functools.cache
def _build():
    info = plsc.get_sparse_core_info()
    nw = info.num_cores * info.num_subcores
    b_per_w = B // nw
    n_chunks = b_per_w // CHUNK
    mesh = plsc.VectorSubcoreMesh(core_axis_name="c", subcore_axis_name="s")

    @functools.partial(
        pl.kernel,
        mesh=mesh,
        out_type=jax.ShapeDtypeStruct((B, D), jnp.float32),
        compiler_params=pltpu.CompilerParams(use_tc_tiling_on_sc=False),
        scratch_types=[
            pltpu.VMEM((b_per_w,), jnp.int32),
            pltpu.VMEM((b_per_w, D), jnp.float32),
            pltpu.SemaphoreType.DMA,
        ],
    )
    def gather_kernel(idx_hbm, table_hbm, out_hbm, idx_v, rows_v, sem):
        wid = lax.axis_index("s") * info.num_cores + lax.axis_index("c")
        base = wid * b_per_w
        pltpu.sync_copy(idx_hbm.at[pl.ds(base, b_per_w)], idx_v)

        copies = [
            pltpu.async_copy(
                table_hbm.at[idx_v.at[pl.ds(c * CHUNK, CHUNK)]],
                rows_v.at[pl.ds(c * CHUNK, CHUNK)],
                sem,
            )
            for c in range(n_chunks)
        ]
        for c, cp in enumerate(copies):
            cp.wait()
            pltpu.sync_copy(
                rows_v.at[pl.ds(c * CHUNK, CHUNK)],
                out_hbm.at[pl.ds(base + c * CHUNK, CHUNK)],
            )

    return gather_kernel


def kernel(indices, weight):
    return _build()(indices.astype(jnp.int32), weight)


# speedup vs baseline: 2.1415x; 2.1398x over previous
import functools

import jax
import jax.numpy as jnp
from jax import lax
from jax.experimental import pallas as pl
from jax.experimental.pallas import tpu as pltpu
from jax.experimental.pallas import tpu_sc as plsc

D = 64
B = 16384
N = 1000000
CHUNK = 128
TBLK = 8192


def _transpose_body(x_ref, o_ref):
    o_ref[...] = x_ref[...].T


@functools.cache
def _transpose_tc():
    grid = (N + TBLK - 1) // TBLK
    return pl.pallas_call(
        _transpose_body,
        grid=(grid,),
        in_specs=[pl.BlockSpec((D, TBLK), lambda i: (0, i))],
        out_specs=pl.BlockSpec((TBLK, D), lambda i: (i, 0)),
        out_shape=jax.ShapeDtypeStruct((N, D), jnp.float32),
    )


@functools.cache
def _gather_sc():
    info = plsc.get_sparse_core_info()
    nw = info.num_cores * info.num_subcores
    b_per_w = B // nw
    n_chunks = b_per_w // CHUNK
    mesh = plsc.VectorSubcoreMesh(core_axis_name="c", subcore_axis_name="s")

    @functools.partial(
        pl.kernel,
        mesh=mesh,
        out_type=jax.ShapeDtypeStruct((B, D), jnp.float32),
        scratch_types=[
            pltpu.VMEM((b_per_w,), jnp.int32),
            pltpu.VMEM((b_per_w, D), jnp.float32),
            pltpu.SemaphoreType.DMA,
        ],
    )
    def gather_kernel(idx_hbm, table_hbm, out_hbm, idx_v, rows_v, sem):
        wid = lax.axis_index("s") * info.num_cores + lax.axis_index("c")
        base = wid * b_per_w
        pltpu.sync_copy(idx_hbm.at[pl.ds(base, b_per_w)], idx_v)

        def issue(g, _):
            vec = idx_v[pl.ds(g * 16, 16)]
            for l in range(16):
                s = vec[l]
                pltpu.async_copy(
                    table_hbm.at[pl.ds(s, 1)],
                    rows_v.at[pl.ds(g * 16 + l, 1)],
                    sem,
                )
            return 0

        for c in range(n_chunks):
            lax.fori_loop(c * CHUNK // 16, (c + 1) * CHUNK // 16, issue, 0)
            pltpu.make_async_copy(
                table_hbm.at[pl.ds(0, CHUNK)],
                rows_v.at[pl.ds(c * CHUNK, CHUNK)],
                sem,
            ).wait()
        pltpu.sync_copy(rows_v, out_hbm.at[pl.ds(base, b_per_w)])

    return gather_kernel


def kernel(indices, weight):
    table = _transpose_tc()(weight.T)
    return _gather_sc()(indices.astype(jnp.int32), table)
